# baseline (speedup 1.0000x reference)
import functools
import jax
import jax.numpy as jnp
from jax import lax
from jax.experimental import pallas as pl
from jax.experimental.pallas import tpu as pltpu
from jax.experimental.pallas import tpu_sc as plsc

NB = 16384
NH = 50
D = 64
NW = 32
BW = NB // NW
HB = 256
L = 16

_mesh = plsc.VectorSubcoreMesh(core_axis_name="c", subcore_axis_name="s")


@functools.partial(
    pl.kernel, mesh=_mesh,
    out_type=jax.ShapeDtypeStruct((NH, D, NB), jnp.float32),
    scratch_types=[
        pltpu.VMEM((HB,), jnp.int32),
        pltpu.VMEM((HB,), jnp.int32),
        pltpu.VMEM((HB,), jnp.int32),
        pltpu.VMEM((HB, 128), jnp.float32),
        pltpu.VMEM((D, HB), jnp.float32),
        pltpu.SemaphoreType.DMA,
        pltpu.SemaphoreType.DMA,
        pltpu.SemaphoreType.DMA,
    ],
    compiler_params=pltpu.CompilerParams(use_tc_tiling_on_sc=True,
                                         needs_layout_passes=False),
)
def _emb2(w128, idx_t, out_hbm,
          idx_v, row_v, par_v, g_v, t_v,
          isem, gsem, ssem):
    wid = lax.axis_index("s") * 2 + lax.axis_index("c")
    b0 = wid * BW

    def stage(s, carry):
        h = s // 2
        cols = b0 + (s % 2) * HB
        pltpu.async_copy(idx_t.at[h, pl.ds(cols, HB)], idx_v, isem).wait()

        def split(k, c):
            v = idx_v[pl.ds(k * L, L)]
            row_v[pl.ds(k * L, L)] = v >> 1
            par_v[pl.ds(k * L, L)] = (v & 1) << 6
            return c

        lax.fori_loop(0, HB // L, split, 0)
        pltpu.async_copy(w128.at[row_v], g_v, gsem).wait()

        def xb(k, c):
            rows16 = jax.lax.broadcasted_iota(jnp.int32, (L,), 0) + k * L
            base = par_v[pl.ds(k * L, L)]
            for d in range(D):
                t_v[d, pl.ds(k * L, L)] = plsc.load_gather(
                    g_v, [rows16, base + d])
            return c

        lax.fori_loop(0, HB // L, xb, 0)
        pltpu.async_copy(t_v, out_hbm.at[h, :, pl.ds(cols, HB)], ssem).wait()
        return carry

    lax.fori_loop(0, NH * 2, stage, 0)


def kernel(input_, weight):
    w128 = weight.reshape(500000, 128)
    idx_t = input_.T
    out = _emb2(w128, idx_t)
    return out.transpose(2, 0, 1)


# speedup vs baseline: 1.6178x; 1.6178x over previous
import functools
import jax
import jax.numpy as jnp
from jax import lax
from jax.experimental import pallas as pl
from jax.experimental.pallas import tpu as pltpu
from jax.experimental.pallas import tpu_sc as plsc

NB = 16384
NH = 50
D = 64
NW = 32
BW = NB // NW
HB = 256
L = 16
NST = NH * 2

_mesh = plsc.VectorSubcoreMesh(core_axis_name="c", subcore_axis_name="s")


@functools.partial(
    pl.kernel, mesh=_mesh,
    out_type=jax.ShapeDtypeStruct((NH, D, NB), jnp.float32),
    scratch_types=[
        pltpu.VMEM((HB,), jnp.int32),
        pltpu.VMEM((HB,), jnp.int32),
        pltpu.VMEM((HB,), jnp.int32),
        pltpu.VMEM((HB,), jnp.int32),
        pltpu.VMEM((HB,), jnp.int32),
        pltpu.VMEM((HB,), jnp.int32),
        pltpu.VMEM((HB, 128), jnp.float32),
        pltpu.VMEM((HB, 128), jnp.float32),
        pltpu.VMEM((D, HB), jnp.float32),
        pltpu.VMEM((D, HB), jnp.float32),
        pltpu.SemaphoreType.DMA,
        pltpu.SemaphoreType.DMA,
        pltpu.SemaphoreType.DMA,
        pltpu.SemaphoreType.DMA,
        pltpu.SemaphoreType.DMA,
        pltpu.SemaphoreType.DMA,
    ],
    compiler_params=pltpu.CompilerParams(use_tc_tiling_on_sc=True,
                                         needs_layout_passes=False),
)
def _emb2(w128, idx_t, out_hbm,
          idx0, idx1, row0, row1, par0, par1, g0, g1, t0, t1,
          isem0, isem1, gsem0, gsem1, ssem0, ssem1):
    idx_v = (idx0, idx1)
    row_v = (row0, row1)
    par_v = (par0, par1)
    g_v = (g0, g1)
    t_v = (t0, t1)
    isems = (isem0, isem1)
    gsems = (gsem0, gsem1)
    ssems = (ssem0, ssem1)

    wid = lax.axis_index("s") * 2 + lax.axis_index("c")
    b0 = wid * BW

    def idx_load(h, p, b):
        pltpu.async_copy(idx_t.at[h, pl.ds(b0 + p * HB, HB)],
                         idx_v[b], isems[b])

    def split_and_gather(b):
        pltpu.make_async_copy(idx_t.at[0, pl.ds(0, HB)], idx_v[b],
                              isems[b]).wait()
        for k in range(HB // L):
            v = idx_v[b][pl.ds(k * L, L)]
            row_v[b][pl.ds(k * L, L)] = v >> 1
            par_v[b][pl.ds(k * L, L)] = (v & 1) << 6
        pltpu.async_copy(w128.at[row_v[b]], g_v[b], gsems[b])

    def extract(b):
        def xb(k, c):
            rows16 = jax.lax.broadcasted_iota(jnp.int32, (L,), 0) + k * L
            base = par_v[b][pl.ds(k * L, L)]
            for d0 in range(0, D, 8):
                vals = [plsc.load_gather(g_v[b], [rows16, base + d])
                        for d in range(d0, d0 + 8)]
                for i in range(8):
                    t_v[b][d0 + i, pl.ds(k * L, L)] = vals[i]
            return c
        lax.fori_loop(0, HB // L, xb, 0)

    idx_load(0, 0, 0)
    idx_load(0, 1, 1)
    split_and_gather(0)
    idx_load(1, 0, 0)

    def body(i, carry):
        @pl.when(i >= 1)
        def _():
            pltpu.make_async_copy(t_v[0], out_hbm.at[0, :, pl.ds(b0, HB)],
                                  ssems[0]).wait()
        split_and_gather(1)
        @pl.when(i + 1 < NH)
        def _():
            idx_load(i + 1, 1, 1)
        pltpu.make_async_copy(w128.at[row_v[0]], g_v[0], gsems[0]).wait()
        extract(0)
        pltpu.async_copy(t_v[0], out_hbm.at[i, :, pl.ds(b0, HB)], ssems[0])

        @pl.when(i >= 1)
        def _():
            pltpu.make_async_copy(t_v[1], out_hbm.at[0, :, pl.ds(b0, HB)],
                                  ssems[1]).wait()
        @pl.when(i + 1 < NH)
        def _():
            split_and_gather(0)
        @pl.when(i + 2 < NH)
        def _():
            idx_load(i + 2, 0, 0)
        pltpu.make_async_copy(w128.at[row_v[1]], g_v[1], gsems[1]).wait()
        extract(1)
        pltpu.async_copy(t_v[1], out_hbm.at[i, :, pl.ds(b0 + HB, HB)],
                         ssems[1])
        return carry

    lax.fori_loop(0, NH, body, 0)

    for b in range(2):
        pltpu.make_async_copy(t_v[b], out_hbm.at[0, :, pl.ds(b0, HB)],
                              ssems[b]).wait()


def kernel(input_, weight):
    w128 = weight.reshape(500000, 128)
    idx_t = input_.T
    out = _emb2(w128, idx_t)
    return out.transpose(2, 0, 1)
